# baseline (speedup 1.0000x reference)
import functools

import jax
import jax.numpy as jnp
from jax import lax
from jax.experimental import pallas as pl
from jax.experimental.pallas import tpu as pltpu
from jax.experimental.pallas import tpu_sc as plsc

N_NODES = 10000
N_EDGES = 320000
D_FEAT = 128
HIDDEN = 128
CLASSES = 40

NCORES = 2
NSUB = 16
NW = NCORES * NSUB
CHUNK = 128
NBUF = 2
IB = 4
NT = -(-N_EDGES // (NW * CHUNK * IB))
N_CHUNKS = NT * IB
E_PAD = N_CHUNKS * NW * CHUNK
DUMMY = N_NODES
ZROWS = 632
N_PAD = NSUB * ZROWS
OROWS = ZROWS
W1 = D_FEAT
W2 = 48
WC = 16


@functools.lru_cache(maxsize=None)
def _make_seg_sum(width, count_cols):
    mesh = plsc.VectorSubcoreMesh(core_axis_name="c", subcore_axis_name="s")
    aw = width + count_cols

    scratch_types = [
        pltpu.VMEM((2, IB, 1, CHUNK), jnp.int32),
        pltpu.VMEM((2, IB, 1, CHUNK), jnp.int32),
        pltpu.VMEM((NBUF, CHUNK, width), jnp.float32),
        pltpu.VMEM_SHARED((N_PAD, width), jnp.float32),
        pltpu.SemaphoreType.DMA,
        pltpu.SemaphoreType.DMA,
    ]
    if count_cols:
        scratch_types += [
            pltpu.VMEM((CHUNK, count_cols), jnp.float32),
            pltpu.VMEM((CHUNK, count_cols), jnp.float32),
            pltpu.VMEM_SHARED((N_PAD, count_cols), jnp.float32),
        ]

    @functools.partial(
        pl.kernel,
        mesh=mesh,
        compiler_params=pltpu.CompilerParams(use_tc_tiling_on_sc=False),
        out_type=jax.ShapeDtypeStruct((NCORES, N_PAD, aw), jnp.float32),
        scratch_types=scratch_types,
    )
    def seg_sum(feat_hbm, src_hbm, dst_hbm, *rest):
        if count_cols:
            (ones_hbm, out_hbm, src_v, dst_v, rows_v, acc, sem, isem,
             ones_v, z8_v, cntr) = rest
        else:
            out_hbm, src_v, dst_v, rows_v, acc, sem, isem = rest
        c = lax.axis_index("c")
        s = lax.axis_index("s")
        wid = s * NCORES + c

        def zrow(r, carry):
            for k in range(width // 16):
                rows_v[0, r, pl.ds(k * 16, 16)] = jnp.zeros(
                    (16,), jnp.float32)
            return carry
        lax.fori_loop(0, CHUNK, zrow, 0)
        if count_cols:
            pltpu.sync_copy(ones_hbm.at[0], ones_v)
            pltpu.sync_copy(ones_hbm.at[1], z8_v)
        row0 = s * ZROWS
        nfull = ZROWS // CHUNK
        def zcopy(t, carry):
            r0 = row0 + t * CHUNK
            pltpu.sync_copy(rows_v.at[0], acc.at[pl.ds(r0, CHUNK)])
            if count_cols:
                pltpu.sync_copy(z8_v, cntr.at[pl.ds(r0, CHUNK)])
            return carry
        lax.fori_loop(0, nfull, zcopy, 0)
        rem = ZROWS - nfull * CHUNK
        if rem:
            r0 = row0 + nfull * CHUNK
            pltpu.sync_copy(rows_v.at[0, pl.ds(0, rem)],
                            acc.at[pl.ds(r0, rem)])
            if count_cols:
                pltpu.sync_copy(z8_v.at[pl.ds(0, rem)],
                                cntr.at[pl.ds(r0, rem)])
        plsc.subcore_barrier()

        def idx_block(t):
            tb = lax.rem(t, 2)
            return (
                pltpu.make_async_copy(src_hbm.at[wid, pl.ds(t * IB, IB)],
                                      src_v.at[tb], isem),
                pltpu.make_async_copy(dst_hbm.at[wid, pl.ds(t * IB, IB)],
                                      dst_v.at[tb], isem),
            )

        def gather(j):
            tb = lax.rem(lax.div(j, IB), 2)
            return pltpu.make_async_copy(
                feat_hbm.at[src_v.at[tb, lax.rem(j, IB), 0]],
                rows_v.at[lax.rem(j, NBUF)], sem)

        for d in idx_block(0):
            d.start()
        for d in idx_block(0):
            d.wait()
        gather(0).start()

        def body(j, carry):
            t = lax.div(j, IB)
            jj = lax.rem(j, IB)
            @pl.when((jj == 0) & (t + 1 < NT))
            def _start_next_idx():
                for d in idx_block(t + 1):
                    d.start()
            @pl.when((jj == IB - 2) & (t + 1 < NT))
            def _wait_next_idx():
                for d in idx_block(t + 1):
                    d.wait()
            gather(j).wait()
            @pl.when(j + 1 < N_CHUNKS)
            def _prefetch():
                gather(j + 1).start()
            tb = lax.rem(t, 2)
            pltpu.sync_copy(rows_v.at[lax.rem(j, NBUF)],
                            acc.at[dst_v.at[tb, jj, 0]], add=True)
            if count_cols:
                pltpu.sync_copy(ones_v, cntr.at[dst_v.at[tb, jj, 0]],
                                add=True)
            return carry
        lax.fori_loop(0, N_CHUNKS, body, 0)
        plsc.subcore_barrier()

        o0 = s * OROWS
        if count_cols:
            pltpu.sync_copy(acc.at[pl.ds(o0, OROWS)],
                            out_hbm.at[c, pl.ds(o0, OROWS), pl.ds(0, width)])
            pltpu.sync_copy(
                cntr.at[pl.ds(o0, OROWS)],
                out_hbm.at[c, pl.ds(o0, OROWS), pl.ds(width, count_cols)])
        else:
            pltpu.sync_copy(acc.at[pl.ds(o0, OROWS)],
                            out_hbm.at[c, pl.ds(o0, OROWS)])

    return seg_sum


def _dense1_body(p0_ref, p1_ref, x_ref, wl1_ref, bl1_ref,
                 wr1_ref, wl2_ref, h_ref, g_ref, ic_ref):
    p0 = p0_ref[...]
    p1 = p1_ref[...]
    ssum = p0[:, :D_FEAT] + p1[:, :D_FEAT]
    cnt = jnp.sum(p0[:, D_FEAT:] + p1[:, D_FEAT:], axis=1,
                  keepdims=True) * (1.0 / WC)
    invc = 1.0 / jnp.maximum(cnt, 1.0)
    mean = ssum * invc
    x = x_ref[...]
    h = jnp.maximum(
        jnp.dot(mean, wl1_ref[...], preferred_element_type=jnp.float32)
        + bl1_ref[...]
        + jnp.dot(x, wr1_ref[...], preferred_element_type=jnp.float32),
        0.0)
    h_ref[...] = h
    g_ref[...] = jnp.dot(h, wl2_ref[...], preferred_element_type=jnp.float32)
    ic_ref[...] = jnp.broadcast_to(invc, ic_ref.shape)


def _dense2_body(q0_ref, q1_ref, ic_ref, h_ref, wr2_ref, bl2_ref, o_ref):
    z = ((q0_ref[...] + q1_ref[...]) * ic_ref[...] + bl2_ref[...]
         + jnp.dot(h_ref[...], wr2_ref[...],
                   preferred_element_type=jnp.float32))
    col = lax.broadcasted_iota(jnp.int32, z.shape, 1)
    valid = col < CLASSES
    zm = jnp.where(valid, z, -1e30)
    m = jnp.max(zm, axis=1, keepdims=True)
    ez = jnp.where(valid, jnp.exp(z - m), 0.0)
    ls = jnp.log(jnp.sum(ez, axis=1, keepdims=True))
    o_ref[...] = (z - m - ls)[:, :CLASSES]


_ROWS_BLK = 1000


def _dense1(p0, p1, x, wl1t, bl1, wr1t, wl2tp):
    grid = (N_NODES // _ROWS_BLK,)
    return pl.pallas_call(
        _dense1_body,
        grid=grid,
        in_specs=[
            pl.BlockSpec((_ROWS_BLK, W1 + WC), lambda i: (i, 0)),
            pl.BlockSpec((_ROWS_BLK, W1 + WC), lambda i: (i, 0)),
            pl.BlockSpec((_ROWS_BLK, D_FEAT), lambda i: (i, 0)),
            pl.BlockSpec((D_FEAT, HIDDEN), lambda i: (0, 0)),
            pl.BlockSpec((1, HIDDEN), lambda i: (0, 0)),
            pl.BlockSpec((D_FEAT, HIDDEN), lambda i: (0, 0)),
            pl.BlockSpec((HIDDEN, W2), lambda i: (0, 0)),
        ],
        out_specs=[
            pl.BlockSpec((_ROWS_BLK, HIDDEN), lambda i: (i, 0)),
            pl.BlockSpec((_ROWS_BLK, W2), lambda i: (i, 0)),
            pl.BlockSpec((_ROWS_BLK, W2), lambda i: (i, 0)),
        ],
        out_shape=[
            jax.ShapeDtypeStruct((N_NODES, HIDDEN), jnp.float32),
            jax.ShapeDtypeStruct((N_NODES, W2), jnp.float32),
            jax.ShapeDtypeStruct((N_NODES, W2), jnp.float32),
        ],
    )(p0, p1, x, wl1t, bl1, wr1t, wl2tp)


def _dense2(q0, q1, ic, h, wr2tp, bl2p):
    grid = (N_NODES // _ROWS_BLK,)
    return pl.pallas_call(
        _dense2_body,
        grid=grid,
        in_specs=[
            pl.BlockSpec((_ROWS_BLK, W2), lambda i: (i, 0)),
            pl.BlockSpec((_ROWS_BLK, W2), lambda i: (i, 0)),
            pl.BlockSpec((_ROWS_BLK, W2), lambda i: (i, 0)),
            pl.BlockSpec((_ROWS_BLK, HIDDEN), lambda i: (i, 0)),
            pl.BlockSpec((HIDDEN, W2), lambda i: (0, 0)),
            pl.BlockSpec((1, W2), lambda i: (0, 0)),
        ],
        out_specs=pl.BlockSpec((_ROWS_BLK, CLASSES), lambda i: (i, 0)),
        out_shape=jax.ShapeDtypeStruct((N_NODES, CLASSES), jnp.float32),
    )(q0, q1, ic, h, wr2tp, bl2p)


def kernel(x, edge_index, Wl1, bl1, Wr1, Wl2, bl2, Wr2):
    x = x.astype(jnp.float32)
    src = edge_index[0].astype(jnp.int32)
    dst = edge_index[1].astype(jnp.int32)
    pad = E_PAD - N_EDGES
    srcp = jnp.concatenate(
        [src, jnp.zeros((pad,), jnp.int32)]).reshape(NW, N_CHUNKS, 1, CHUNK)
    dstp = jnp.concatenate(
        [dst, jnp.full((pad,), DUMMY, jnp.int32)]).reshape(NW, N_CHUNKS, 1, CHUNK)

    xa = jnp.concatenate([x, jnp.ones((N_NODES, WC), jnp.float32)], axis=1)
    part1 = _make_seg_sum(W1 + WC, 0)(xa, srcp, dstp)[:, :N_NODES]

    wl1t = Wl1.T
    wr1t = Wr1.T
    wl2tp = jnp.pad(Wl2.T, ((0, 0), (0, W2 - CLASSES)))
    h, g48, ic48 = _dense1(part1[0], part1[1], x, wl1t,
                           bl1.reshape(1, HIDDEN), wr1t, wl2tp)

    part2 = _make_seg_sum(W2, 0)(g48, srcp, dstp)[:, :N_NODES]

    wr2tp = jnp.pad(Wr2.T, ((0, 0), (0, W2 - CLASSES)))
    bl2p = jnp.pad(bl2, (0, W2 - CLASSES)).reshape(1, W2)
    return _dense2(part2[0], part2[1], ic48, h, wr2tp, bl2p)


# speedup vs baseline: 1.9755x; 1.9755x over previous
import functools

import jax
import jax.numpy as jnp
from jax import lax
from jax.experimental import pallas as pl
from jax.experimental.pallas import tpu as pltpu
from jax.experimental.pallas import tpu_sc as plsc

N_NODES = 10000
N_EDGES = 320000
D_FEAT = 128
HIDDEN = 128
CLASSES = 40

NCORES = 2
NSUB = 16
NW = NCORES * NSUB
CH1 = 72
NC1 = -(-N_EDGES // (NW * CH1))
EP1 = NC1 * NW * CH1
CH2 = 128
NC2 = -(-N_EDGES // (NW * CH2))
EP2 = NC2 * NW * CH2
DUMMY = N_NODES
ZROWS = 626
N_PAD = NSUB * ZROWS
OROWS = ZROWS
W1 = D_FEAT
W2 = 48
WC = 16


@functools.lru_cache(maxsize=None)
def _make_seg_sum(width, count_cols, chunk, n_chunks):
    mesh = plsc.VectorSubcoreMesh(core_axis_name="c", subcore_axis_name="s")
    aw = width + count_cols

    scratch_types = [
        pltpu.VMEM((n_chunks, 1, chunk), jnp.int32),
        pltpu.VMEM((n_chunks, 1, chunk), jnp.int32),
        pltpu.VMEM((2, chunk, width), jnp.float32),
        pltpu.VMEM_SHARED((N_PAD, width), jnp.float32),
        pltpu.SemaphoreType.DMA,
    ]
    if count_cols:
        scratch_types += [
            pltpu.VMEM((chunk, count_cols), jnp.float32),
            pltpu.VMEM((chunk, count_cols), jnp.float32),
            pltpu.VMEM_SHARED((N_PAD, count_cols), jnp.float32),
        ]

    @functools.partial(
        pl.kernel,
        mesh=mesh,
        compiler_params=pltpu.CompilerParams(use_tc_tiling_on_sc=False),
        out_type=jax.ShapeDtypeStruct((NCORES, N_PAD, aw), jnp.float32),
        scratch_types=scratch_types,
    )
    def seg_sum(feat_hbm, src_hbm, dst_hbm, *rest):
        if count_cols:
            (ones_hbm, out_hbm, src_v, dst_v, rows_v, acc, sem,
             ones_v, z8_v, cntr) = rest
        else:
            out_hbm, src_v, dst_v, rows_v, acc, sem = rest
        c = lax.axis_index("c")
        s = lax.axis_index("s")
        wid = s * NCORES + c

        def zrow(r, carry):
            for k in range(width // 16):
                rows_v[0, r, pl.ds(k * 16, 16)] = jnp.zeros(
                    (16,), jnp.float32)
            return carry
        lax.fori_loop(0, chunk, zrow, 0)
        if count_cols:
            pltpu.sync_copy(ones_hbm.at[0], ones_v)
            pltpu.sync_copy(ones_hbm.at[1], z8_v)
        row0 = s * ZROWS
        nfull = ZROWS // chunk
        def zcopy(t, carry):
            r0 = row0 + t * chunk
            pltpu.sync_copy(rows_v.at[0], acc.at[pl.ds(r0, chunk)])
            if count_cols:
                pltpu.sync_copy(z8_v, cntr.at[pl.ds(r0, chunk)])
            return carry
        lax.fori_loop(0, nfull, zcopy, 0)
        rem = ZROWS - nfull * chunk
        if rem:
            r0 = row0 + nfull * chunk
            pltpu.sync_copy(rows_v.at[0, pl.ds(0, rem)],
                            acc.at[pl.ds(r0, rem)])
            if count_cols:
                pltpu.sync_copy(z8_v.at[pl.ds(0, rem)],
                                cntr.at[pl.ds(r0, rem)])
        plsc.subcore_barrier()

        pltpu.sync_copy(src_hbm.at[wid], src_v)
        pltpu.sync_copy(dst_hbm.at[wid], dst_v)

        def gather(j, b):
            return pltpu.make_async_copy(feat_hbm.at[src_v.at[j, 0]],
                                         rows_v.at[b], sem)
        gather(0, 0).start()
        def body(j, carry):
            b = lax.rem(j, 2)
            gather(j, b).wait()
            @pl.when(j + 1 < n_chunks)
            def _prefetch():
                gather(j + 1, 1 - b).start()
            pltpu.sync_copy(rows_v.at[b], acc.at[dst_v.at[j, 0]], add=True)
            if count_cols:
                pltpu.sync_copy(ones_v, cntr.at[dst_v.at[j, 0]], add=True)
            return carry
        lax.fori_loop(0, n_chunks, body, 0)
        plsc.subcore_barrier()

        o0 = s * OROWS
        if count_cols:
            pltpu.sync_copy(acc.at[pl.ds(o0, OROWS)],
                            out_hbm.at[c, pl.ds(o0, OROWS), pl.ds(0, width)])
            pltpu.sync_copy(
                cntr.at[pl.ds(o0, OROWS)],
                out_hbm.at[c, pl.ds(o0, OROWS), pl.ds(width, count_cols)])
        else:
            pltpu.sync_copy(acc.at[pl.ds(o0, OROWS)],
                            out_hbm.at[c, pl.ds(o0, OROWS)])

    return seg_sum


def _dense1_body(p0_ref, p1_ref, x_ref, wl1_ref, bl1_ref,
                 wr1_ref, wl2_ref, h_ref, g_ref, ic_ref):
    p0 = p0_ref[...]
    p1 = p1_ref[...]
    ssum = p0[:, :D_FEAT] + p1[:, :D_FEAT]
    cnt = jnp.sum(p0[:, D_FEAT:] + p1[:, D_FEAT:], axis=1,
                  keepdims=True) * (1.0 / WC)
    invc = 1.0 / jnp.maximum(cnt, 1.0)
    mean = ssum * invc
    x = x_ref[...]
    h = jnp.maximum(
        jnp.dot(mean, wl1_ref[...], preferred_element_type=jnp.float32)
        + bl1_ref[...]
        + jnp.dot(x, wr1_ref[...], preferred_element_type=jnp.float32),
        0.0)
    h_ref[...] = h
    g_ref[...] = jnp.dot(h, wl2_ref[...], preferred_element_type=jnp.float32)
    ic_ref[...] = jnp.broadcast_to(invc, ic_ref.shape)


def _dense2_body(q0_ref, q1_ref, ic_ref, h_ref, wr2_ref, bl2_ref, o_ref):
    z = ((q0_ref[...] + q1_ref[...]) * ic_ref[...] + bl2_ref[...]
         + jnp.dot(h_ref[...], wr2_ref[...],
                   preferred_element_type=jnp.float32))
    col = lax.broadcasted_iota(jnp.int32, z.shape, 1)
    valid = col < CLASSES
    zm = jnp.where(valid, z, -1e30)
    m = jnp.max(zm, axis=1, keepdims=True)
    ez = jnp.where(valid, jnp.exp(z - m), 0.0)
    ls = jnp.log(jnp.sum(ez, axis=1, keepdims=True))
    o_ref[...] = (z - m - ls)[:, :CLASSES]


_ROWS_BLK = 1000


def _dense1(p0, p1, x, wl1t, bl1, wr1t, wl2tp):
    grid = (N_NODES // _ROWS_BLK,)
    return pl.pallas_call(
        _dense1_body,
        grid=grid,
        in_specs=[
            pl.BlockSpec((_ROWS_BLK, W1 + WC), lambda i: (i, 0)),
            pl.BlockSpec((_ROWS_BLK, W1 + WC), lambda i: (i, 0)),
            pl.BlockSpec((_ROWS_BLK, D_FEAT), lambda i: (i, 0)),
            pl.BlockSpec((D_FEAT, HIDDEN), lambda i: (0, 0)),
            pl.BlockSpec((1, HIDDEN), lambda i: (0, 0)),
            pl.BlockSpec((D_FEAT, HIDDEN), lambda i: (0, 0)),
            pl.BlockSpec((HIDDEN, W2), lambda i: (0, 0)),
        ],
        out_specs=[
            pl.BlockSpec((_ROWS_BLK, HIDDEN), lambda i: (i, 0)),
            pl.BlockSpec((_ROWS_BLK, W2), lambda i: (i, 0)),
            pl.BlockSpec((_ROWS_BLK, W2), lambda i: (i, 0)),
        ],
        out_shape=[
            jax.ShapeDtypeStruct((N_NODES, HIDDEN), jnp.float32),
            jax.ShapeDtypeStruct((N_NODES, W2), jnp.float32),
            jax.ShapeDtypeStruct((N_NODES, W2), jnp.float32),
        ],
    )(p0, p1, x, wl1t, bl1, wr1t, wl2tp)


def _dense2(q0, q1, ic, h, wr2tp, bl2p):
    grid = (N_NODES // _ROWS_BLK,)
    return pl.pallas_call(
        _dense2_body,
        grid=grid,
        in_specs=[
            pl.BlockSpec((_ROWS_BLK, W2), lambda i: (i, 0)),
            pl.BlockSpec((_ROWS_BLK, W2), lambda i: (i, 0)),
            pl.BlockSpec((_ROWS_BLK, W2), lambda i: (i, 0)),
            pl.BlockSpec((_ROWS_BLK, HIDDEN), lambda i: (i, 0)),
            pl.BlockSpec((HIDDEN, W2), lambda i: (0, 0)),
            pl.BlockSpec((1, W2), lambda i: (0, 0)),
        ],
        out_specs=pl.BlockSpec((_ROWS_BLK, CLASSES), lambda i: (i, 0)),
        out_shape=jax.ShapeDtypeStruct((N_NODES, CLASSES), jnp.float32),
    )(q0, q1, ic, h, wr2tp, bl2p)


def _edge_layout(src, dst, chunk, n_chunks, e_pad):
    pad = e_pad - N_EDGES
    srcp = jnp.concatenate(
        [src, jnp.zeros((pad,), jnp.int32)]).reshape(NW, n_chunks, 1, chunk)
    dstp = jnp.concatenate(
        [dst, jnp.full((pad,), DUMMY, jnp.int32)]).reshape(
            NW, n_chunks, 1, chunk)
    return srcp, dstp


def kernel(x, edge_index, Wl1, bl1, Wr1, Wl2, bl2, Wr2):
    x = x.astype(jnp.float32)
    src = edge_index[0].astype(jnp.int32)
    dst = edge_index[1].astype(jnp.int32)
    srcp1, dstp1 = _edge_layout(src, dst, CH1, NC1, EP1)
    srcp2, dstp2 = _edge_layout(src, dst, CH2, NC2, EP2)

    xa = jnp.concatenate([x, jnp.ones((N_NODES, WC), jnp.float32)], axis=1)
    part1 = _make_seg_sum(W1 + WC, 0, CH1, NC1)(xa, srcp1, dstp1)[:, :N_NODES]

    wl1t = Wl1.T
    wr1t = Wr1.T
    wl2tp = jnp.pad(Wl2.T, ((0, 0), (0, W2 - CLASSES)))
    h, g48, ic48 = _dense1(part1[0], part1[1], x, wl1t,
                           bl1.reshape(1, HIDDEN), wr1t, wl2tp)

    part2 = _make_seg_sum(W2, 0, CH2, NC2)(g48, srcp2, dstp2)[:, :N_NODES]

    wr2tp = jnp.pad(Wr2.T, ((0, 0), (0, W2 - CLASSES)))
    bl2p = jnp.pad(bl2, (0, W2 - CLASSES)).reshape(1, W2)
    return _dense2(part2[0], part2[1], ic48, h, wr2tp, bl2p)
